# baseline (speedup 1.0000x reference)
import functools

import numpy as np
import jax
import jax.numpy as jnp
from jax import lax
from jax.experimental import pallas as pl
from jax.experimental.pallas import tpu as pltpu
from jax.experimental.pallas import tpu_sc as plsc

_SIZE = 14
_NH = 16
_L = _SIZE * _SIZE
_NIDX = _L * _L
_TAB = (2 * _SIZE - 1) ** 2
_N = _NH * _NIDX
_NW = 32
_LANES = 16
_NPAD = ((_N + _NW * _LANES - 1) // (_NW * _LANES)) * (_NW * _LANES)
_PER_W = _NPAD // _NW
_NVEC = _PER_W // _LANES


def _rel_pos_index() -> np.ndarray:
    g = np.arange(_SIZE)
    gi, gj = np.meshgrid(g, g, indexing="ij")
    gg = np.stack([gi, gj]).reshape(2, -1)
    coords = gg[:, :, None] - gg[:, None, :] + (_SIZE - 1)
    coords[0] *= 2 * _SIZE - 1
    return coords.sum(0)


_FLAT_IDX = np.zeros((_NPAD,), np.int32)
_FLAT_IDX[:_N] = (np.arange(_NH, dtype=np.int64)[:, None] * _TAB
                  + _rel_pos_index().reshape(-1)[None, :]).reshape(-1)


@functools.cache
def _sc_gather_kernel():
    @functools.partial(
        pl.kernel,
        mesh=plsc.VectorSubcoreMesh(core_axis_name="c", subcore_axis_name="s"),
        out_type=jax.ShapeDtypeStruct((_NPAD,), jnp.float32),
        scratch_types=[
            pltpu.VMEM((_NH * _TAB,), jnp.float32),
            pltpu.VMEM((_PER_W,), jnp.int32),
            pltpu.VMEM((_PER_W,), jnp.float32),
        ],
        compiler_params=pltpu.CompilerParams(needs_layout_passes=False),
    )
    def _sc_gather(table_hbm, idx_hbm, out_hbm, tab_v, idx_v, vals_v):
        wid = lax.axis_index("s") * 2 + lax.axis_index("c")
        base = wid * _PER_W
        pltpu.sync_copy(table_hbm, tab_v)
        pltpu.sync_copy(idx_hbm.at[pl.ds(base, _PER_W)], idx_v)

        def body(i, carry):
            iv = idx_v[pl.ds(i * _LANES, _LANES)]
            vals_v[pl.ds(i * _LANES, _LANES)] = plsc.load_gather(tab_v, [iv])
            return carry

        lax.fori_loop(0, _NVEC, body, 0)
        pltpu.sync_copy(vals_v, out_hbm.at[pl.ds(base, _PER_W)])

    return _sc_gather


_BB = 8
_NSPLIT = 2
_BN = _N // _NSPLIT


def _add_body(x_ref, b_ref, o_ref):
    o_ref[...] = x_ref[...] + b_ref[...]


def kernel(x, weight):
    bias_flat = _sc_gather_kernel()(weight.reshape(-1), jnp.asarray(_FLAT_IDX))
    bias2 = bias_flat[:_N].reshape(1, _N)
    x2 = x.reshape(x.shape[0], _N)
    out = pl.pallas_call(
        _add_body,
        grid=(x.shape[0] // _BB, _NSPLIT),
        in_specs=[
            pl.BlockSpec((_BB, _BN), lambda i, j: (i, j)),
            pl.BlockSpec((1, _BN), lambda i, j: (0, j)),
        ],
        out_specs=pl.BlockSpec((_BB, _BN), lambda i, j: (i, j)),
        out_shape=jax.ShapeDtypeStruct(x2.shape, x2.dtype),
    )(x2, bias2)
    return out.reshape(x.shape)


# speedup vs baseline: 1.9716x; 1.9716x over previous
import functools

import numpy as np
import jax
import jax.numpy as jnp
from jax import lax
from jax.experimental import pallas as pl
from jax.experimental.pallas import tpu as pltpu
from jax.experimental.pallas import tpu_sc as plsc

_SIZE = 14
_NH = 16
_L = _SIZE * _SIZE
_NIDX = _L * _L
_TAB = (2 * _SIZE - 1) ** 2
_N = _NH * _NIDX
_NW = 32
_LANES = 16
_NPAD = ((_N + _NW * _LANES - 1) // (_NW * _LANES)) * (_NW * _LANES)
_PER_W = _NPAD // _NW
_NVEC = _PER_W // _LANES


def _rel_pos_index() -> np.ndarray:
    g = np.arange(_SIZE)
    gi, gj = np.meshgrid(g, g, indexing="ij")
    gg = np.stack([gi, gj]).reshape(2, -1)
    coords = gg[:, :, None] - gg[:, None, :] + (_SIZE - 1)
    coords[0] *= 2 * _SIZE - 1
    return coords.sum(0)


_FLAT_IDX = np.zeros((_NPAD,), np.int32)
_FLAT_IDX[:_N] = (np.arange(_NH, dtype=np.int64)[:, None] * _TAB
                  + _rel_pos_index().reshape(-1)[None, :]).reshape(-1)


@functools.cache
def _sc_gather_kernel():
    @functools.partial(
        pl.kernel,
        mesh=plsc.VectorSubcoreMesh(core_axis_name="c", subcore_axis_name="s"),
        out_type=jax.ShapeDtypeStruct((_NPAD,), jnp.float32),
        scratch_types=[
            pltpu.VMEM((_NH * _TAB,), jnp.float32),
            pltpu.VMEM((_PER_W,), jnp.int32),
            pltpu.VMEM((_PER_W,), jnp.float32),
        ],
        compiler_params=pltpu.CompilerParams(needs_layout_passes=False),
    )
    def _sc_gather(table_hbm, idx_hbm, out_hbm, tab_v, idx_v, vals_v):
        wid = lax.axis_index("s") * 2 + lax.axis_index("c")
        base = wid * _PER_W
        pltpu.sync_copy(table_hbm, tab_v)
        pltpu.sync_copy(idx_hbm.at[pl.ds(base, _PER_W)], idx_v)

        def body(i, carry):
            iv = idx_v[pl.ds(i * _LANES, _LANES)]
            vals_v[pl.ds(i * _LANES, _LANES)] = plsc.load_gather(tab_v, [iv])
            return carry

        lax.fori_loop(0, _NVEC, body, 0)
        pltpu.sync_copy(vals_v, out_hbm.at[pl.ds(base, _PER_W)])

    return _sc_gather


_BB = 2


def _add_body(x_ref, b_ref, o_ref):
    o_ref[...] = x_ref[...] + b_ref[...]


def kernel(x, weight):
    bias_flat = _sc_gather_kernel()(weight.reshape(-1), jnp.asarray(_FLAT_IDX))
    bias3 = bias_flat[:_N].reshape(_NH, _L, _L)
    out = pl.pallas_call(
        _add_body,
        grid=(x.shape[0] // _BB,),
        in_specs=[
            pl.BlockSpec((_BB, _NH, _L, _L), lambda i: (i, 0, 0, 0)),
            pl.BlockSpec((_NH, _L, _L), lambda i: (0, 0, 0)),
        ],
        out_specs=pl.BlockSpec((_BB, _NH, _L, _L), lambda i: (i, 0, 0, 0)),
        out_shape=jax.ShapeDtypeStruct(x.shape, x.dtype),
    )(x, bias3)
    return out


# speedup vs baseline: 6.0056x; 3.0461x over previous
import functools

import numpy as np
import jax
import jax.numpy as jnp
from jax import lax
from jax.experimental import pallas as pl
from jax.experimental.pallas import tpu as pltpu
from jax.experimental.pallas import tpu_sc as plsc

_SIZE = 14
_NH = 16
_L = _SIZE * _SIZE
_NIDX = _L * _L
_TAB = (2 * _SIZE - 1) ** 2
_N = _NH * _NIDX
_NW = 32
_LANES = 16
_NPAD = ((_N + _NW * _LANES - 1) // (_NW * _LANES)) * (_NW * _LANES)
_PER_W = _NPAD // _NW
_NVEC = _PER_W // _LANES


def _rel_pos_index() -> np.ndarray:
    g = np.arange(_SIZE)
    gi, gj = np.meshgrid(g, g, indexing="ij")
    gg = np.stack([gi, gj]).reshape(2, -1)
    coords = gg[:, :, None] - gg[:, None, :] + (_SIZE - 1)
    coords[0] *= 2 * _SIZE - 1
    return coords.sum(0)


_FLAT_IDX = np.zeros((_NPAD,), np.int32)
_FLAT_IDX[:_N] = (np.arange(_NH, dtype=np.int64)[None, :, None] * _TAB
                  + _rel_pos_index()[:, None, :]).reshape(-1)


@functools.cache
def _sc_gather_kernel():
    @functools.partial(
        pl.kernel,
        mesh=plsc.VectorSubcoreMesh(core_axis_name="c", subcore_axis_name="s"),
        out_type=jax.ShapeDtypeStruct((_NPAD,), jnp.float32),
        scratch_types=[
            pltpu.VMEM((_NH * _TAB,), jnp.float32),
            pltpu.VMEM((_PER_W,), jnp.int32),
            pltpu.VMEM((_PER_W,), jnp.float32),
        ],
        compiler_params=pltpu.CompilerParams(needs_layout_passes=False),
    )
    def _sc_gather(table_hbm, idx_hbm, out_hbm, tab_v, idx_v, vals_v):
        wid = lax.axis_index("s") * 2 + lax.axis_index("c")
        base = wid * _PER_W
        pltpu.sync_copy(table_hbm, tab_v)
        pltpu.sync_copy(idx_hbm.at[pl.ds(base, _PER_W)], idx_v)

        def body(i, carry):
            iv = idx_v[pl.ds(i * _LANES, _LANES)]
            vals_v[pl.ds(i * _LANES, _LANES)] = plsc.load_gather(tab_v, [iv])
            return carry

        lax.fori_loop(0, _NVEC, body, 0)
        pltpu.sync_copy(vals_v, out_hbm.at[pl.ds(base, _PER_W)])

    return _sc_gather


_BB = 4


def _add_body(x_ref, b_ref, o_ref):
    o_ref[...] = x_ref[...] + b_ref[...]


def kernel(x, weight):
    bias_flat = _sc_gather_kernel()(weight.reshape(-1), jnp.asarray(_FLAT_IDX))
    bias3 = bias_flat[:_N].reshape(_L, _NH, _L)
    xt = jnp.transpose(x, (0, 2, 1, 3))
    out = pl.pallas_call(
        _add_body,
        grid=(x.shape[0] // _BB,),
        in_specs=[
            pl.BlockSpec((_BB, _L, _NH, _L), lambda i: (i, 0, 0, 0)),
            pl.BlockSpec((_L, _NH, _L), lambda i: (0, 0, 0)),
        ],
        out_specs=pl.BlockSpec((_BB, _L, _NH, _L), lambda i: (i, 0, 0, 0)),
        out_shape=jax.ShapeDtypeStruct(xt.shape, xt.dtype),
    )(xt, bias3)
    return jnp.transpose(out, (0, 2, 1, 3))
